# baseline (speedup 1.0000x reference)
import functools

import jax
import jax.numpy as jnp
import numpy as np
from jax import lax
from jax.experimental import pallas as pl
from jax.experimental.pallas import tpu as pltpu
from jax.experimental.pallas import tpu_sc as plsc

D_MODEL = 1024
H_TOTAL = 32
H_ACTIVE = 4
D_HEAD = D_MODEL // H_TOTAL
HG = 8
_BT = 256


def _logits_body(x_ref, wg_ref, o_ref):
    o_ref[...] = jax.lax.dot_general(
        wg_ref[...], x_ref[...], (((0,), (1,)), ((), ())),
        preferred_element_type=jnp.float32)


def _make_sc_router(t):
    info = plsc.get_sparse_core_info()
    tok_per_w = 128
    nchunks = t // tok_per_w
    ngrp = tok_per_w // 16
    mesh = plsc.VectorSubcoreMesh(core_axis_name="c", subcore_axis_name="s")

    import functools as _ft

    def _router_chunk(logits_hbm, w_hbm, lg_v, w_v, wid):
        base = wid * tok_per_w
        pltpu.sync_copy(logits_hbm.at[:, pl.ds(base, tok_per_w)], lg_v)
        neg = jnp.full((16,), -1e30, jnp.float32)

        def per_group(g, carry):
            c0 = g * 16
            m = neg
            for h in range(H_TOTAL):
                m = jnp.maximum(m, lg_v[h, pl.ds(c0, 16)])
            tot = jnp.zeros((16,), jnp.float32)
            for h in range(H_TOTAL):
                tot = tot + jnp.exp(lg_v[h, pl.ds(c0, 16)] - m)
            sel = []
            val = []
            for r in range(H_ACTIVE):
                cur = neg
                idx = jnp.full((16,), H_TOTAL, jnp.int32)
                for h in range(H_TOTAL):
                    lh = lg_v[h, pl.ds(c0, 16)]
                    active = jnp.full((16,), True)
                    for pr in range(r):
                        active = active & (sel[pr] != h)
                    take = active & (lh > cur)
                    cur = jnp.where(take, lh, cur)
                    idx = jnp.where(take, jnp.int32(h), idx)
                sel.append(idx)
                val.append(jnp.exp(cur - m) / tot)
            for h in range(H_TOTAL):
                wh = jnp.zeros((16,), jnp.float32)
                for r in range(H_ACTIVE):
                    wh = wh + jnp.where(sel[r] == h, val[r],
                                        jnp.float32(0.0))
                w_v[h, pl.ds(c0, 16)] = wh
            return carry

        lax.fori_loop(0, ngrp, per_group, 0)
        pltpu.sync_copy(w_v, w_hbm.at[:, pl.ds(base, tok_per_w)])

    @_ft.partial(
        pl.kernel, mesh=mesh,
        out_type=jax.ShapeDtypeStruct((H_TOTAL, t), jnp.float32),
        scratch_types=[
            pltpu.VMEM((H_TOTAL, tok_per_w), jnp.float32),
            pltpu.VMEM((H_TOTAL, tok_per_w), jnp.float32),
        ],
    )
    def sc_router(logits_hbm, w_hbm, lg_v, w_v):
        wid = lax.axis_index("s") * info.num_cores + lax.axis_index("c")

        @pl.when(wid < nchunks)
        def _work():
            _router_chunk(logits_hbm, w_hbm, lg_v, w_v, wid)

    return sc_router


def _matmul_body(x_ref, w_ref, o_ref):
    o_ref[...] = jnp.dot(x_ref[...], w_ref[...],
                         preferred_element_type=jnp.float32)


def _qkv_body(x_ref, w_ref, o_ref):
    o_ref[...] = jnp.dot(x_ref[...], w_ref[...],
                         preferred_element_type=jnp.float32
                         ).astype(jnp.bfloat16)


def _attn_body(q_ref, k_ref, v_ref, g_ref, o_ref, *, scale, t, off):
    g = pl.program_id(0)
    i = pl.program_id(1) + off
    q_blk = q_ref[...]
    sel = (jax.lax.broadcasted_iota(jnp.int32, (H_TOTAL, HG), 0)
           == g * HG + jax.lax.broadcasted_iota(jnp.int32, (H_TOTAL, HG), 1)
           ).astype(jnp.float32)
    gates = jax.lax.dot_general(g_ref[...], sel, (((0,), (0,)), ((), ())),
                                preferred_element_type=jnp.float32)

    row = i * _BT + jax.lax.broadcasted_iota(jnp.int32, (_BT, t), 0)
    col = jax.lax.broadcasted_iota(jnp.int32, (_BT, t), 1)
    msk = col <= row
    neg = jnp.float32(-1e9)

    for hh in range(HG):
        hs = slice(hh * D_HEAD, (hh + 1) * D_HEAD)
        q = q_blk[:, hs]
        k = k_ref[:, hs]
        s = jax.lax.dot_general(q, k, (((1,), (1,)), ((), ())),
                                preferred_element_type=jnp.float32)
        s = jnp.where(msk, s * scale, neg)
        m = jnp.max(s, axis=-1, keepdims=True)
        p = jnp.exp(s - m)
        l = jnp.sum(p, axis=-1, keepdims=True)
        out = jnp.dot(p.astype(jnp.bfloat16), v_ref[:, hs],
                      preferred_element_type=jnp.float32)
        o_ref[:, hs] = ((out / l) * gates[:, hh:hh + 1]).astype(jnp.bfloat16)


@jax.jit
def kernel(x, Wg, Wq, Wk, Wv, Wo):
    b, t, d = x.shape
    x2 = x.reshape(t, d)

    logits = pl.pallas_call(
        _logits_body,
        grid=(1,),
        in_specs=[
            pl.BlockSpec((t, d), lambda i: (0, 0)),
            pl.BlockSpec((d, H_TOTAL), lambda i: (0, 0)),
        ],
        out_specs=pl.BlockSpec((H_TOTAL, t), lambda i: (0, 0)),
        out_shape=jax.ShapeDtypeStruct((H_TOTAL, t), jnp.float32),
    )(x2, Wg)
    w = _make_sc_router(t)(logits)

    xb = x2.astype(jnp.bfloat16)
    wqkv = jnp.concatenate([Wq, Wk, Wv], axis=1).astype(jnp.bfloat16)
    bn = 512
    bm = 512
    qkv = pl.pallas_call(
        _qkv_body,
        grid=(t // bm, 3 * d // bn),
        in_specs=[
            pl.BlockSpec((bm, d), lambda i, j: (i, 0)),
            pl.BlockSpec((d, bn), lambda i, j: (0, j)),
        ],
        out_specs=pl.BlockSpec((bm, bn), lambda i, j: (i, j)),
        out_shape=jax.ShapeDtypeStruct((t, 3 * d), jnp.bfloat16),
    )(xb, wqkv)

    scale = np.float32(1.0 / np.sqrt(D_HEAD))
    ng = H_TOTAL // HG
    wide = HG * D_HEAD
    qb_per_call = 1
    pieces = []
    for c in range(t // _BT // qb_per_call):
        off = c * qb_per_call
        kw = (off + qb_per_call) * _BT
        pieces.append(pl.pallas_call(
            functools.partial(_attn_body, scale=scale, t=kw, off=off),
            grid=(ng, qb_per_call),
            in_specs=[
                pl.BlockSpec((_BT, wide),
                             lambda g, i, off=off: (off + i, g)),
                pl.BlockSpec((kw, wide), lambda g, i: (0, ng + g)),
                pl.BlockSpec((kw, wide), lambda g, i: (0, 2 * ng + g)),
                pl.BlockSpec((H_TOTAL, _BT),
                             lambda g, i, off=off: (0, off + i)),
            ],
            out_specs=pl.BlockSpec((_BT, wide), lambda g, i: (i, g)),
            out_shape=jax.ShapeDtypeStruct((qb_per_call * _BT, d),
                                           jnp.bfloat16),
        )(qkv, qkv, qkv, w))
    attn_out = jnp.concatenate(pieces, axis=0)

    y = pl.pallas_call(
        _matmul_body,
        grid=(d // bn,),
        in_specs=[
            pl.BlockSpec((t, d), lambda j: (0, 0)),
            pl.BlockSpec((d, bn), lambda j: (0, j)),
        ],
        out_specs=pl.BlockSpec((t, bn), lambda j: (0, j)),
        out_shape=jax.ShapeDtypeStruct((t, d), jnp.float32),
    )(attn_out, Wo.astype(jnp.bfloat16))

    return y.reshape(b, t, d)


# speedup vs baseline: 1.0628x; 1.0628x over previous
import functools

import jax
import jax.numpy as jnp
import numpy as np
from jax import lax
from jax.experimental import pallas as pl
from jax.experimental.pallas import tpu as pltpu
from jax.experimental.pallas import tpu_sc as plsc

D_MODEL = 1024
H_TOTAL = 32
H_ACTIVE = 4
D_HEAD = D_MODEL // H_TOTAL
HG = 8
_BT = 256


def _logits_body(x_ref, wg_ref, o_ref):
    o_ref[...] = jax.lax.dot_general(
        wg_ref[...], x_ref[...], (((0,), (1,)), ((), ())),
        preferred_element_type=jnp.float32)


def _make_sc_router(t):
    info = plsc.get_sparse_core_info()
    tok_per_w = 128
    nchunks = t // tok_per_w
    ngrp = tok_per_w // 16
    mesh = plsc.VectorSubcoreMesh(core_axis_name="c", subcore_axis_name="s")

    import functools as _ft

    def _router_chunk(logits_hbm, w_hbm, lg_v, w_v, wid):
        base = wid * tok_per_w
        pltpu.sync_copy(logits_hbm.at[:, pl.ds(base, tok_per_w)], lg_v)
        neg = jnp.full((16,), -1e30, jnp.float32)

        def per_group(g, carry):
            c0 = g * 16
            m = neg
            for h in range(H_TOTAL):
                m = jnp.maximum(m, lg_v[h, pl.ds(c0, 16)])
            tot = jnp.zeros((16,), jnp.float32)
            for h in range(H_TOTAL):
                tot = tot + jnp.exp(lg_v[h, pl.ds(c0, 16)] - m)
            sel = []
            val = []
            for r in range(H_ACTIVE):
                cur = neg
                idx = jnp.full((16,), H_TOTAL, jnp.int32)
                for h in range(H_TOTAL):
                    lh = lg_v[h, pl.ds(c0, 16)]
                    active = jnp.full((16,), True)
                    for pr in range(r):
                        active = active & (sel[pr] != h)
                    take = active & (lh > cur)
                    cur = jnp.where(take, lh, cur)
                    idx = jnp.where(take, jnp.int32(h), idx)
                sel.append(idx)
                val.append(jnp.exp(cur - m) / tot)
            for h in range(H_TOTAL):
                wh = jnp.zeros((16,), jnp.float32)
                for r in range(H_ACTIVE):
                    wh = wh + jnp.where(sel[r] == h, val[r],
                                        jnp.float32(0.0))
                w_v[h, pl.ds(c0, 16)] = wh
            return carry

        lax.fori_loop(0, ngrp, per_group, 0)
        pltpu.sync_copy(w_v, w_hbm.at[:, pl.ds(base, tok_per_w)])

    @_ft.partial(
        pl.kernel, mesh=mesh,
        out_type=jax.ShapeDtypeStruct((H_TOTAL, t), jnp.float32),
        scratch_types=[
            pltpu.VMEM((H_TOTAL, tok_per_w), jnp.float32),
            pltpu.VMEM((H_TOTAL, tok_per_w), jnp.float32),
        ],
    )
    def sc_router(logits_hbm, w_hbm, lg_v, w_v):
        wid = lax.axis_index("s") * info.num_cores + lax.axis_index("c")

        @pl.when(wid < nchunks)
        def _work():
            _router_chunk(logits_hbm, w_hbm, lg_v, w_v, wid)

    return sc_router


def _matmul_body(x_ref, w_ref, o_ref):
    o_ref[...] = jnp.dot(x_ref[...], w_ref[...].astype(jnp.bfloat16),
                         preferred_element_type=jnp.float32)


def _qkv_body(x_ref, w_ref, o_ref):
    o_ref[...] = jnp.dot(x_ref[...].astype(jnp.bfloat16),
                         w_ref[...].astype(jnp.bfloat16),
                         preferred_element_type=jnp.float32
                         ).astype(jnp.bfloat16)


def _attn_body(q_ref, k_ref, v_ref, g_ref, o_ref, *, scale, t, off):
    g = pl.program_id(0)
    i = pl.program_id(1) + off
    q_blk = q_ref[...]
    sel = (jax.lax.broadcasted_iota(jnp.int32, (H_TOTAL, HG), 0)
           == g * HG + jax.lax.broadcasted_iota(jnp.int32, (H_TOTAL, HG), 1)
           ).astype(jnp.float32)
    gates = jax.lax.dot_general(g_ref[...], sel, (((0,), (0,)), ((), ())),
                                preferred_element_type=jnp.float32)

    row = i * _BT + jax.lax.broadcasted_iota(jnp.int32, (_BT, t), 0)
    col = jax.lax.broadcasted_iota(jnp.int32, (_BT, t), 1)
    msk = col <= row
    neg = jnp.float32(-1e9)

    for hh in range(HG):
        hs = slice(hh * D_HEAD, (hh + 1) * D_HEAD)
        q = q_blk[:, hs]
        k = k_ref[:, hs]
        s = jax.lax.dot_general(q, k, (((1,), (1,)), ((), ())),
                                preferred_element_type=jnp.float32)
        s = jnp.where(msk, s * scale, neg)
        m = jnp.max(s, axis=-1, keepdims=True)
        p = jnp.exp(s - m)
        l = jnp.sum(p, axis=-1, keepdims=True)
        out = jnp.dot(p.astype(jnp.bfloat16), v_ref[:, hs],
                      preferred_element_type=jnp.float32)
        o_ref[:, hs] = ((out / l) * gates[:, hh:hh + 1]).astype(jnp.bfloat16)


@jax.jit
def kernel(x, Wg, Wq, Wk, Wv, Wo):
    b, t, d = x.shape
    x2 = x.reshape(t, d)

    logits = pl.pallas_call(
        _logits_body,
        grid=(1,),
        in_specs=[
            pl.BlockSpec((t, d), lambda i: (0, 0)),
            pl.BlockSpec((d, H_TOTAL), lambda i: (0, 0)),
        ],
        out_specs=pl.BlockSpec((H_TOTAL, t), lambda i: (0, 0)),
        out_shape=jax.ShapeDtypeStruct((H_TOTAL, t), jnp.float32),
    )(x2, Wg)
    w = _make_sc_router(t)(logits)

    xb = x2
    wqkv = jnp.concatenate([Wq, Wk, Wv], axis=1)
    bn = 512
    qkv = pl.pallas_call(
        _qkv_body,
        grid=(3 * d // bn,),
        in_specs=[
            pl.BlockSpec((t, d), lambda j: (0, 0)),
            pl.BlockSpec((d, bn), lambda j: (0, j)),
        ],
        out_specs=pl.BlockSpec((t, bn), lambda j: (0, j)),
        out_shape=jax.ShapeDtypeStruct((t, 3 * d), jnp.bfloat16),
    )(xb, wqkv)

    scale = np.float32(1.0 / np.sqrt(D_HEAD))
    ng = H_TOTAL // HG
    wide = HG * D_HEAD
    qb_per_call = 1
    pieces = []
    for c in range(t // _BT // qb_per_call):
        off = c * qb_per_call
        kw = (off + qb_per_call) * _BT
        pieces.append(pl.pallas_call(
            functools.partial(_attn_body, scale=scale, t=kw, off=off),
            grid=(ng, qb_per_call),
            in_specs=[
                pl.BlockSpec((_BT, wide),
                             lambda g, i, off=off: (off + i, g)),
                pl.BlockSpec((kw, wide), lambda g, i: (0, ng + g)),
                pl.BlockSpec((kw, wide), lambda g, i: (0, 2 * ng + g)),
                pl.BlockSpec((H_TOTAL, _BT),
                             lambda g, i, off=off: (0, off + i)),
            ],
            out_specs=pl.BlockSpec((_BT, wide), lambda g, i: (i, g)),
            out_shape=jax.ShapeDtypeStruct((qb_per_call * _BT, d),
                                           jnp.bfloat16),
        )(qkv, qkv, qkv, w))
    attn_out = jnp.concatenate(pieces, axis=0)

    y = pl.pallas_call(
        _matmul_body,
        grid=(d // bn,),
        in_specs=[
            pl.BlockSpec((t, d), lambda j: (0, 0)),
            pl.BlockSpec((d, bn), lambda j: (0, j)),
        ],
        out_specs=pl.BlockSpec((t, bn), lambda j: (0, j)),
        out_shape=jax.ShapeDtypeStruct((t, d), jnp.float32),
    )(attn_out, Wo)

    return y.reshape(b, t, d)
